# baseline (speedup 1.0000x reference)
import functools

import jax
import jax.numpy as jnp
import numpy as np
from jax import lax
from jax.experimental import pallas as pl
from jax.experimental.pallas import tpu as pltpu
from jax.experimental.pallas import tpu_sc as plsc

HASH_LOGSIZE = 20
DIM = 32
BATCH = 16384
NCORES = 2
NSUB = 16
NW = NCORES * NSUB
BPW = BATCH // NW
BLK = 128
NBLK = BPW // BLK
CLIP = 127.0 / 128.0
HASHM = np.int32(np.uint32(2654435761).astype(np.int64) - (1 << 32))

_CORNERS = [(0, 4, 0, 4, 0), (0, 4, 3, 7, 1), (3, 7, 3, 7, 2), (3, 7, 0, 4, 3)]
_P3 = 3 ** np.arange(16, dtype=np.int64)
CORNER_CELLS = []
for (_y0, _y1, _x0, _x1, _k) in _CORNERS:
    _ids = np.arange(16).reshape(4, 4)
    _flat = np.rot90(_ids, _k).reshape(-1)
    _cells = []
    for _i, _cid in enumerate(_flat):
        _r, _c = divmod(int(_cid), 4)
        _cells.append(((_y0 + _r) * 7 + (_x0 + _c), int(_P3[_i])))
    CORNER_CELLS.append(_cells)

CHAN_BASE = [(d // 8) * (1 << 23) + (d % 8) * 128 for d in range(DIM)]
QSPAN = 8387712

_MESH = plsc.VectorSubcoreMesh(core_axis_name="c", subcore_axis_name="s")


@functools.partial(
    pl.kernel,
    mesh=_MESH,
    compiler_params=pltpu.CompilerParams(
        use_tc_tiling_on_sc=False, needs_layout_passes=False
    ),
    out_type=jax.ShapeDtypeStruct((BATCH * DIM,), jnp.float32),
    scratch_types=[
        pltpu.VMEM((BPW, 49), jnp.int32),
        pltpu.VMEM((2, 4, BLK), jnp.int32),
        pltpu.VMEM((2, 4, DIM, BLK), jnp.float32),
        pltpu.VMEM((BLK * DIM,), jnp.float32),
        pltpu.SemaphoreType.DMA,
        pltpu.SemaphoreType.DMA,
    ],
)
def _sc_feature(data_hbm, tlin_hbm, feat_hbm, cells_v, q_v, gath_v, feat_v,
                sem0, sem1):
    wid = lax.axis_index("s") * NCORES + lax.axis_index("c")
    base = wid * BPW
    pltpu.sync_copy(data_hbm.at[pl.ds(base, BPW), :], cells_v)
    sems = [sem0, sem1]
    iota16 = lax.iota(jnp.int32, 16)

    def compute_q(j, buf):
        for corner in range(4):
            cellcoef = CORNER_CELLS[corner]

            def u_body(u, carry, corner=corner, cellcoef=cellcoef, buf=buf, j=j):
                rowv = iota16 + (j * BLK + u * 16)
                raw = None
                for cell, coef in cellcoef:
                    colv = jnp.full((16,), cell, jnp.int32)
                    term = plsc.load_gather(cells_v, [rowv, colv]) * np.int32(coef)
                    raw = term if raw is None else raw + term
                i = lax.shift_right_logical(raw * HASHM, 12)
                q = lax.shift_right_logical(i, 7) * 1024 + (i & 127)
                q_v[buf, corner, pl.ds(u * 16, 16)] = q
                return carry

            lax.fori_loop(0, BLK // 16, u_body, 0)

    def fire(j, buf):
        for corner in range(4):
            for d in range(DIM):
                pltpu.async_copy(
                    tlin_hbm.at[pl.ds(CHAN_BASE[d], QSPAN)].at[
                        q_v.at[buf, corner]
                    ],
                    gath_v.at[buf, corner, d],
                    sems[buf],
                )

    def drain(buf):
        def w_body(k, carry, buf=buf):
            pltpu.make_async_copy(
                tlin_hbm.at[pl.ds(0, QSPAN)].at[q_v.at[buf, 0]],
                gath_v.at[buf, 0, 0],
                sems[buf],
            ).wait()
            return carry

        lax.fori_loop(0, 4 * DIM, w_body, 0)

    def sum_out(j, buf):
        iota32 = iota16 * DIM

        def m_body(mg, carry, buf=buf):
            for d in range(DIM):
                sl = pl.ds(mg * 16, 16)
                acc = (
                    gath_v[buf, 0, d, sl]
                    + gath_v[buf, 1, d, sl]
                    + gath_v[buf, 2, d, sl]
                    + gath_v[buf, 3, d, sl]
                )
                plsc.store_scatter(feat_v, [iota32 + (mg * 16 * DIM + d)], acc)
            return carry

        lax.fori_loop(0, BLK // 16, m_body, 0)
        pltpu.sync_copy(
            feat_v, feat_hbm.at[pl.ds((base + j * BLK) * DIM, BLK * DIM)]
        )

    compute_q(0, 0)
    fire(0, 0)
    for j in range(1, NBLK):
        buf, prev = j % 2, (j - 1) % 2
        compute_q(j, buf)
        fire(j, buf)
        drain(prev)
        sum_out(j - 1, prev)
    drain((NBLK - 1) % 2)
    sum_out(NBLK - 1, (NBLK - 1) % 2)


def _mlp_body(f_ref, w1_ref, b1_ref, w2_ref, b2_ref, w3_ref, b3_ref, o_ref):
    x = jnp.clip(f_ref[...], -1.0, CLIP)
    dn = (((1,), (1,)), ((), ()))
    h = lax.dot_general(x, w1_ref[...], dn, preferred_element_type=jnp.float32)
    h = jnp.clip(h + b1_ref[...], 0.0, CLIP)
    h = lax.dot_general(h, w2_ref[...], dn, preferred_element_type=jnp.float32)
    h = jnp.clip(h + b2_ref[...], 0.0, CLIP)
    o_ref[...] = (
        lax.dot_general(h, w3_ref[...], dn, preferred_element_type=jnp.float32)
        + b3_ref[...]
    )


def _mlp(feat, W1, b1, W2, b2, W3p, b3p):
    blk = 2048
    return pl.pallas_call(
        _mlp_body,
        grid=(BATCH // blk,),
        in_specs=[
            pl.BlockSpec((blk, DIM), lambda i: (i, 0)),
            pl.BlockSpec((DIM, DIM), lambda i: (0, 0)),
            pl.BlockSpec((1, DIM), lambda i: (0, 0)),
            pl.BlockSpec((DIM, DIM), lambda i: (0, 0)),
            pl.BlockSpec((1, DIM), lambda i: (0, 0)),
            pl.BlockSpec((8, DIM), lambda i: (0, 0)),
            pl.BlockSpec((1, 8), lambda i: (0, 0)),
        ],
        out_specs=pl.BlockSpec((blk, 8), lambda i: (i, 0)),
        out_shape=jax.ShapeDtypeStruct((BATCH, 8), jnp.float32),
    )(feat, W1, b1, W2, b2, W3p, b3p)


def kernel(data, table, W1, b1, W2, b2, W3, b3):
    B = data.shape[0]
    tlin = table.reshape(8192, 128, 4, 8).transpose(2, 0, 3, 1).reshape(-1)
    feat = _sc_feature(data.reshape(B, 49), tlin).reshape(B, DIM)
    W3p = jnp.zeros((8, DIM), jnp.float32).at[:3].set(W3)
    b3p = jnp.zeros((1, 8), jnp.float32).at[0, :3].set(b3)
    v = _mlp(feat, W1, b1.reshape(1, DIM), W2, b2.reshape(1, DIM), W3p, b3p)[:, :3]
    policy = jnp.zeros((B, 7, 7), jnp.float32)
    return (v, policy)


# speedup vs baseline: 1.0086x; 1.0086x over previous
import functools

import jax
import jax.numpy as jnp
import numpy as np
from jax import lax
from jax.experimental import pallas as pl
from jax.experimental.pallas import tpu as pltpu
from jax.experimental.pallas import tpu_sc as plsc

HASH_LOGSIZE = 20
DIM = 32
BATCH = 16384
NCORES = 2
NSUB = 16
NW = NCORES * NSUB
BPW = BATCH // NW
BLK = 128
NBLK = BPW // BLK
CLIP = 127.0 / 128.0
HASHM = np.int32(np.uint32(2654435761).astype(np.int64) - (1 << 32))

_CORNERS = [(0, 4, 0, 4, 0), (0, 4, 3, 7, 1), (3, 7, 3, 7, 2), (3, 7, 0, 4, 3)]
_P3 = 3 ** np.arange(16, dtype=np.int64)
CORNER_CELLS = []
for (_y0, _y1, _x0, _x1, _k) in _CORNERS:
    _ids = np.arange(16).reshape(4, 4)
    _flat = np.rot90(_ids, _k).reshape(-1)
    _cells = []
    for _i, _cid in enumerate(_flat):
        _r, _c = divmod(int(_cid), 4)
        _cells.append(((_y0 + _r) * 7 + (_x0 + _c), int(_P3[_i])))
    CORNER_CELLS.append(_cells)

CHAN_BASE = [(d // 8) * (1 << 23) + (d % 8) * 128 for d in range(DIM)]
QSPAN = 8387712

_MESH = plsc.VectorSubcoreMesh(core_axis_name="c", subcore_axis_name="s")


@functools.partial(
    pl.kernel,
    mesh=_MESH,
    compiler_params=pltpu.CompilerParams(
        use_tc_tiling_on_sc=False, needs_layout_passes=False
    ),
    out_type=jax.ShapeDtypeStruct((BATCH * DIM,), jnp.float32),
    scratch_types=[
        pltpu.VMEM((BPW, 64), jnp.int32),
        pltpu.VMEM((2, 4, BLK), jnp.int32),
        pltpu.VMEM((2, 4, DIM, BLK), jnp.float32),
        pltpu.VMEM((BLK * DIM,), jnp.float32),
        pltpu.SemaphoreType.DMA,
        pltpu.SemaphoreType.DMA,
    ],
)
def _sc_feature(data_hbm, tlin_hbm, feat_hbm, cells_v, q_v, gath_v, feat_v,
                sem0, sem1):
    wid = lax.axis_index("s") * NCORES + lax.axis_index("c")
    base = wid * BPW
    pltpu.sync_copy(data_hbm.at[pl.ds(base, BPW), :], cells_v)
    sems = [sem0, sem1]
    iota16 = lax.iota(jnp.int32, 16)

    def compute_q(j, buf):
        for corner in range(4):
            cellcoef = CORNER_CELLS[corner]

            def u_body(u, carry, corner=corner, cellcoef=cellcoef, buf=buf, j=j):
                rowv = iota16 + (j * BLK + u * 16)
                raw = None
                for cell, coef in cellcoef:
                    colv = jnp.full((16,), cell, jnp.int32)
                    term = plsc.load_gather(cells_v, [rowv, colv]) * np.int32(coef)
                    raw = term if raw is None else raw + term
                i = lax.shift_right_logical(raw * HASHM, 12)
                q = lax.shift_right_logical(i, 7) * 1024 + (i & 127)
                q_v[buf, corner, pl.ds(u * 16, 16)] = q
                return carry

            lax.fori_loop(0, BLK // 16, u_body, 0)

    def fire(j, buf):
        for corner in range(4):
            for d in range(DIM):
                pltpu.async_copy(
                    tlin_hbm.at[pl.ds(CHAN_BASE[d], QSPAN)].at[
                        q_v.at[buf, corner]
                    ],
                    gath_v.at[buf, corner, d],
                    sems[buf],
                )

    def drain(buf):
        def w_body(k, carry, buf=buf):
            pltpu.make_async_copy(
                tlin_hbm.at[pl.ds(0, QSPAN)].at[q_v.at[buf, 0]],
                gath_v.at[buf, 0, 0],
                sems[buf],
            ).wait()
            return carry

        lax.fori_loop(0, 4 * DIM, w_body, 0)

    def sum_out(j, buf):
        iota32 = iota16 * DIM

        def m_body(mg, carry, buf=buf):
            for d in range(DIM):
                sl = pl.ds(mg * 16, 16)
                acc = (
                    gath_v[buf, 0, d, sl]
                    + gath_v[buf, 1, d, sl]
                    + gath_v[buf, 2, d, sl]
                    + gath_v[buf, 3, d, sl]
                )
                plsc.store_scatter(feat_v, [iota32 + (mg * 16 * DIM + d)], acc)
            return carry

        lax.fori_loop(0, BLK // 16, m_body, 0)
        pltpu.sync_copy(
            feat_v, feat_hbm.at[pl.ds((base + j * BLK) * DIM, BLK * DIM)]
        )

    compute_q(0, 0)
    fire(0, 0)
    for j in range(1, NBLK):
        buf, prev = j % 2, (j - 1) % 2
        compute_q(j, buf)
        fire(j, buf)
        drain(prev)
        sum_out(j - 1, prev)
    drain((NBLK - 1) % 2)
    sum_out(NBLK - 1, (NBLK - 1) % 2)


def _mlp_body(f_ref, w1_ref, b1_ref, w2_ref, b2_ref, w3_ref, b3_ref, o_ref):
    x = jnp.clip(f_ref[...], -1.0, CLIP)
    dn = (((1,), (0,)), ((), ()))
    h = lax.dot_general(x, w1_ref[...], dn, preferred_element_type=jnp.float32)
    h = jnp.clip(h + b1_ref[...], 0.0, CLIP)
    h = lax.dot_general(h, w2_ref[...], dn, preferred_element_type=jnp.float32)
    h = jnp.clip(h + b2_ref[...], 0.0, CLIP)
    o_ref[...] = (
        lax.dot_general(h, w3_ref[...], dn, preferred_element_type=jnp.float32)
        + b3_ref[...]
    )


def _mlp(f2d, W1d, b1d, W2d, b2d, W3d, b3d):
    rows = BATCH * DIM // 128
    blk = 1024
    return pl.pallas_call(
        _mlp_body,
        grid=(rows // blk,),
        in_specs=[
            pl.BlockSpec((blk, 128), lambda i: (i, 0)),
            pl.BlockSpec((128, 128), lambda i: (0, 0)),
            pl.BlockSpec((1, 128), lambda i: (0, 0)),
            pl.BlockSpec((128, 128), lambda i: (0, 0)),
            pl.BlockSpec((1, 128), lambda i: (0, 0)),
            pl.BlockSpec((128, 128), lambda i: (0, 0)),
            pl.BlockSpec((1, 128), lambda i: (0, 0)),
        ],
        out_specs=pl.BlockSpec((blk, 128), lambda i: (i, 0)),
        out_shape=jax.ShapeDtypeStruct((rows, 128), jnp.float32),
    )(f2d, W1d, b1d, W2d, b2d, W3d, b3d)


def _blockdiag4(w):
    z = jnp.zeros((128, 128), jnp.float32)
    for sblk in range(4):
        z = z.at[sblk * 32:(sblk + 1) * 32, sblk * 32:(sblk + 1) * 32].set(w)
    return z


def kernel(data, table, W1, b1, W2, b2, W3, b3):
    B = data.shape[0]
    tlin = table.reshape(8192, 128, 4, 8).transpose(2, 0, 3, 1).reshape(-1)
    dataP = jnp.pad(data.reshape(B, 49), ((0, 0), (0, 15)))
    f2d = _sc_feature(dataP, tlin).reshape(BATCH * DIM // 128, 128)
    W1d = _blockdiag4(W1.T)
    W2d = _blockdiag4(W2.T)
    W3d = _blockdiag4(jnp.zeros((DIM, DIM), jnp.float32).at[:, :3].set(W3.T))
    b1d = jnp.tile(b1, 4).reshape(1, 128)
    b2d = jnp.tile(b2, 4).reshape(1, 128)
    b3d = jnp.tile(jnp.zeros((DIM,), jnp.float32).at[:3].set(b3), 4).reshape(1, 128)
    out = _mlp(f2d, W1d, b1d, W2d, b2d, W3d, b3d)
    v = out.reshape(BATCH, DIM)[:, :3]
    policy = jnp.zeros((B, 7, 7), jnp.float32)
    return (v, policy)


# speedup vs baseline: 1.2007x; 1.1904x over previous
import functools

import jax
import jax.numpy as jnp
import numpy as np
from jax import lax
from jax.experimental import pallas as pl
from jax.experimental.pallas import tpu as pltpu
from jax.experimental.pallas import tpu_sc as plsc

HASH_LOGSIZE = 20
DIM = 32
BATCH = 16384
NCORES = 2
NSUB = 16
NW = NCORES * NSUB
BPW = BATCH // NW
BLK = 128
NBLK = BPW // BLK
CLIP = 127.0 / 128.0
HASHM = np.int32(np.uint32(2654435761).astype(np.int64) - (1 << 32))

_CORNERS = [(0, 4, 0, 4, 0), (0, 4, 3, 7, 1), (3, 7, 3, 7, 2), (3, 7, 0, 4, 3)]
_P3 = 3 ** np.arange(16, dtype=np.int64)
_COEF = np.zeros((4, 7, 7), dtype=np.int32)
for _ci, (_y0, _y1, _x0, _x1, _k) in enumerate(_CORNERS):
    _ids = np.arange(16).reshape(4, 4)
    _flat = np.rot90(_ids, _k).reshape(-1)
    for _i, _cid in enumerate(_flat):
        _r, _c = divmod(int(_cid), 4)
        _COEF[_ci, _y0 + _r, _x0 + _c] = int(_P3[_i])

CHAN_BASE = [(d // 8) * (1 << 23) + (d % 8) * 128 for d in range(DIM)]
QSPAN = 8387712

_MESH = plsc.VectorSubcoreMesh(core_axis_name="c", subcore_axis_name="s")


@functools.partial(
    pl.kernel,
    mesh=_MESH,
    compiler_params=pltpu.CompilerParams(
        use_tc_tiling_on_sc=False, needs_layout_passes=False
    ),
    out_type=jax.ShapeDtypeStruct((BATCH * DIM,), jnp.float32),
    scratch_types=[
        pltpu.VMEM((4, BPW), jnp.int32),
        pltpu.VMEM((2, 4, DIM, BLK), jnp.float32),
        pltpu.VMEM((BLK * DIM,), jnp.float32),
        pltpu.SemaphoreType.DMA,
        pltpu.SemaphoreType.DMA,
    ],
)
def _sc_feature(q_hbm, tlin_hbm, feat_hbm, q_v, gath_v, feat_v, sem0, sem1):
    wid = lax.axis_index("s") * NCORES + lax.axis_index("c")
    base = wid * BPW
    for corner in range(4):
        pltpu.sync_copy(
            q_hbm.at[pl.ds(corner * BATCH + base, BPW)], q_v.at[corner]
        )
    sems = [sem0, sem1]
    iota16 = lax.iota(jnp.int32, 16)

    def fire(j, buf):
        for corner in range(4):
            qrow = q_v.at[corner, pl.ds(j * BLK, BLK)]
            for d in range(DIM):
                pltpu.async_copy(
                    tlin_hbm.at[pl.ds(CHAN_BASE[d], QSPAN)].at[qrow],
                    gath_v.at[buf, corner, d],
                    sems[buf],
                )

    def drain(buf):
        def w_body(k, carry, buf=buf):
            pltpu.make_async_copy(
                tlin_hbm.at[pl.ds(0, QSPAN)].at[q_v.at[0, pl.ds(0, BLK)]],
                gath_v.at[buf, 0, 0],
                sems[buf],
            ).wait()
            return carry

        lax.fori_loop(0, 4 * DIM, w_body, 0)

    def sum_out(j, buf):
        iota32 = iota16 * DIM

        def m_body(mg, carry, buf=buf):
            for d in range(DIM):
                sl = pl.ds(mg * 16, 16)
                acc = (
                    gath_v[buf, 0, d, sl]
                    + gath_v[buf, 1, d, sl]
                    + gath_v[buf, 2, d, sl]
                    + gath_v[buf, 3, d, sl]
                )
                plsc.store_scatter(feat_v, [iota32 + (mg * 16 * DIM + d)], acc)
            return carry

        lax.fori_loop(0, BLK // 16, m_body, 0)
        pltpu.sync_copy(
            feat_v, feat_hbm.at[pl.ds((base + j * BLK) * DIM, BLK * DIM)]
        )

    fire(0, 0)
    fire(1, 1)
    for j in range(NBLK):
        buf = j % 2
        drain(buf)
        sum_out(j, buf)
        if j + 2 < NBLK:
            fire(j + 2, buf)


def _hash_q(data):
    qs = []
    for c in range(4):
        coef = jnp.asarray(_COEF[c])
        raw = jnp.sum(data * coef[None, :, :], axis=(1, 2))
        i = lax.shift_right_logical(raw * HASHM, 12)
        q = lax.shift_right_logical(i, 7) * 1024 + (i & 127)
        qs.append(q)
    return jnp.concatenate(qs)


def _mlp_body(f_ref, w1_ref, b1_ref, w2_ref, b2_ref, w3_ref, b3_ref, o_ref):
    x = jnp.clip(f_ref[...], -1.0, CLIP)
    dn = (((1,), (0,)), ((), ()))
    h = lax.dot_general(x, w1_ref[...], dn, preferred_element_type=jnp.float32)
    h = jnp.clip(h + b1_ref[...], 0.0, CLIP)
    h = lax.dot_general(h, w2_ref[...], dn, preferred_element_type=jnp.float32)
    h = jnp.clip(h + b2_ref[...], 0.0, CLIP)
    o_ref[...] = (
        lax.dot_general(h, w3_ref[...], dn, preferred_element_type=jnp.float32)
        + b3_ref[...]
    )


def _mlp(f2d, W1d, b1d, W2d, b2d, W3d, b3d):
    rows = BATCH * DIM // 128
    blk = 1024
    return pl.pallas_call(
        _mlp_body,
        grid=(rows // blk,),
        in_specs=[
            pl.BlockSpec((blk, 128), lambda i: (i, 0)),
            pl.BlockSpec((128, 128), lambda i: (0, 0)),
            pl.BlockSpec((1, 128), lambda i: (0, 0)),
            pl.BlockSpec((128, 128), lambda i: (0, 0)),
            pl.BlockSpec((1, 128), lambda i: (0, 0)),
            pl.BlockSpec((128, 128), lambda i: (0, 0)),
            pl.BlockSpec((1, 128), lambda i: (0, 0)),
        ],
        out_specs=pl.BlockSpec((blk, 128), lambda i: (i, 0)),
        out_shape=jax.ShapeDtypeStruct((rows, 128), jnp.float32),
    )(f2d, W1d, b1d, W2d, b2d, W3d, b3d)


def _blockdiag4(w):
    z = jnp.zeros((128, 128), jnp.float32)
    for sblk in range(4):
        z = z.at[sblk * 32:(sblk + 1) * 32, sblk * 32:(sblk + 1) * 32].set(w)
    return z


def kernel(data, table, W1, b1, W2, b2, W3, b3):
    B = data.shape[0]
    tlin = table.reshape(8192, 128, 4, 8).transpose(2, 0, 3, 1).reshape(-1)
    q = _hash_q(data)
    f2d = _sc_feature(q, tlin).reshape(BATCH * DIM // 128, 128)
    W1d = _blockdiag4(W1.T)
    W2d = _blockdiag4(W2.T)
    W3d = _blockdiag4(jnp.zeros((DIM, DIM), jnp.float32).at[:, :3].set(W3.T))
    b1d = jnp.tile(b1, 4).reshape(1, 128)
    b2d = jnp.tile(b2, 4).reshape(1, 128)
    b3d = jnp.tile(jnp.zeros((DIM,), jnp.float32).at[:3].set(b3), 4).reshape(1, 128)
    out = _mlp(f2d, W1d, b1d, W2d, b2d, W3d, b3d)
    v = out.reshape(BATCH, DIM)[:, :3]
    policy = jnp.zeros((B, 7, 7), jnp.float32)
    return (v, policy)
